# baseline (speedup 1.0000x reference)
import functools

import jax
import jax.numpy as jnp
from jax import lax
from jax.experimental import pallas as pl
from jax.experimental.pallas import tpu as pltpu
from jax.experimental.pallas import tpu_sc as plsc

VOCAB = 1000000
D = 32
O = 6
B = 4096
H = 200

NC = 2
NS = 16
L = 16
NW = NC * NS
EPW = B // NW

NBF = VOCAB // 128
TAIL = VOCAB - NBF * 128
ROUNDS = (NBF + NW - 1) // NW

CHUNK = 4
NCH = EPW // CHUNK
GSUB = 200
NG = CHUNK * H // GSUB

_mesh = plsc.VectorSubcoreMesh(
    core_axis_name="c", subcore_axis_name="s", num_cores=NC, num_subcores=NS)


@functools.partial(
    pl.kernel,
    out_type=jax.ShapeDtypeStruct((VOCAB * D,), jnp.float32),
    mesh=_mesh,
    compiler_params=pltpu.CompilerParams(
        needs_layout_passes=False, use_tc_tiling_on_sc=True),
    scratch_types=dict(
        blk_a=pltpu.VMEM((D, 128), jnp.float32),
        blk_b=pltpu.VMEM((D, 128), jnp.float32),
        rows_a=pltpu.VMEM((128 * D,), jnp.float32),
        rows_b=pltpu.VMEM((128 * D,), jnp.float32),
        tail_v=pltpu.VMEM((TAIL * D,), jnp.float32),
        sem_a=pltpu.SemaphoreType.DMA,
        sem_b=pltpu.SemaphoreType.DMA,
        osem_a=pltpu.SemaphoreType.DMA,
        osem_b=pltpu.SemaphoreType.DMA,
    ),
)
def _transpose_kernel(tT_hbm, tail_hbm, out_hbm, blk_a, blk_b, rows_a,
                      rows_b, tail_v, sem_a, sem_b, osem_a, osem_b):
  wid = lax.axis_index("s") * NC + lax.axis_index("c")
  iota = lax.iota(jnp.int32, L)

  def fire(j, blk, sem):
    off = pl.multiple_of(j * 128, 128)
    pltpu.async_copy(tT_hbm.at[:, pl.ds(off, 128)], blk, sem)

  def drain_in(blk, sem):
    pltpu.make_async_copy(tT_hbm.at[:, pl.ds(0, 128)], blk, sem).wait()

  def shuffle(j, blk, rows, osem):
    for rl in range(128):
      for h in range(2):
        v = plsc.load_gather(
            blk, [iota + L * h, jnp.full((L,), rl, jnp.int32)])
        rows[pl.ds(rl * D + L * h, L)] = v
    off = pl.multiple_of(j * (128 * D), 8)
    pltpu.async_copy(rows, out_hbm.at[pl.ds(off, 128 * D)], osem)

  def drain_out(rows, osem):
    pltpu.make_async_copy(rows, out_hbm.at[pl.ds(0, 128 * D)], osem).wait()

  fire(wid, blk_a, sem_a)

  def body(t, _):
    i0 = t * 2
    j0 = i0 * NW + wid
    j1 = j0 + NW
    j2 = j1 + NW
    drain_in(blk_a, sem_a)

    @pl.when(j1 < NBF)
    def _():
      fire(j1, blk_b, sem_b)

    @pl.when(t > 0)
    def _():
      drain_out(rows_a, osem_a)

    shuffle(j0, blk_a, rows_a, osem_a)

    @pl.when(j1 < NBF)
    def _():
      drain_in(blk_b, sem_b)

      @pl.when(j2 < NBF)
      def _():
        fire(j2, blk_a, sem_a)

      @pl.when(t > 0)
      def _():
        drain_out(rows_b, osem_b)

      shuffle(j1, blk_b, rows_b, osem_b)
    return 0

  nit = (ROUNDS + 1) // 2

  def guarded_body(t, c):
    @pl.when(t * 2 * NW + wid < NBF)
    def _():
      body(t, c)
    return 0

  lax.fori_loop(0, nit, guarded_body, 0)

  @pl.when(wid == 0)
  def _():
    pltpu.sync_copy(tail_hbm, tail_v)
    pltpu.sync_copy(tail_v, out_hbm.at[pl.ds(NBF * 128 * D, TAIL * D)])

  drain_out(rows_a, osem_a)
  drain_out(rows_b, osem_b)


@functools.partial(
    pl.kernel,
    out_type=jax.ShapeDtypeStruct((B * O,), jnp.float32),
    mesh=_mesh,
    compiler_params=pltpu.CompilerParams(
        needs_layout_passes=False, use_tc_tiling_on_sc=False),
    scratch_types=dict(
        idx_v=pltpu.VMEM((EPW * H,), jnp.int32),
        rows_a=pltpu.VMEM((CHUNK * H, D), jnp.float32),
        rows_b=pltpu.VMEM((CHUNK * H, D), jnp.float32),
        acc=pltpu.VMEM((EPW * D,), jnp.float32),
        wv=pltpu.VMEM((O, D, L), jnp.float32),
        bv=pltpu.VMEM((O, L), jnp.float32),
        out_v=pltpu.VMEM((EPW * O,), jnp.float32),
        sem_a=pltpu.SemaphoreType.DMA,
        sem_b=pltpu.SemaphoreType.DMA,
    ),
)
def _gather_kernel(x_hbm, table_hbm, w_hbm, b_hbm, out_hbm,
                   idx_v, rows_a, rows_b, acc, wv, bv, out_v, sem_a, sem_b):
  wid = lax.axis_index("s") * NC + lax.axis_index("c")
  base = wid * EPW

  pltpu.sync_copy(x_hbm.at[pl.ds(base * H, EPW * H)], idx_v)
  pltpu.sync_copy(w_hbm, wv)
  pltpu.sync_copy(b_hbm, bv)

  def fire(c, buf, sem):
    for j in range(NG):
      off = pl.multiple_of(c * (CHUNK * H) + j * GSUB, 8)
      pltpu.async_copy(
          table_hbm.at[idx_v.at[pl.ds(off, GSUB)]],
          buf.at[pl.ds(j * GSUB, GSUB)], sem)

  def drain(buf, sem):
    pltpu.make_async_copy(table_hbm.at[pl.ds(0, CHUNK * H)], buf, sem).wait()

  zeros = jnp.zeros((L,), jnp.float32)
  zeros8 = (zeros,) * 8

  def reduce(c, buf):
    for el in range(CHUNK):
      def body(i, carry, el=el):
        c0, c1, c2, c3, c4, c5, c6, c7 = carry
        r = el * H + i * 8
        for k in range(4):
          c0 = c0 + buf[r + 2 * k, pl.ds(0, L)]
          c1 = c1 + buf[r + 2 * k, pl.ds(L, L)]
          c2 = c2 + buf[r + 2 * k + 1, pl.ds(0, L)]
          c3 = c3 + buf[r + 2 * k + 1, pl.ds(L, L)]
        return (c2, c3, c4, c5, c6, c7, c0, c1)
      ch = lax.fori_loop(0, H // 8, body, zeros8)
      a0 = (ch[0] + ch[2]) + (ch[4] + ch[6])
      a1 = (ch[1] + ch[3]) + (ch[5] + ch[7])
      eoff = pl.multiple_of((c * CHUNK + el) * D, 8)
      acc[pl.ds(eoff, L)] = a0
      acc[pl.ds(eoff + L, L)] = a1

  fire(0, rows_a, sem_a)

  def pipe(ee, _):
    c0 = ee * 2
    drain(rows_a, sem_a)
    fire(c0 + 1, rows_b, sem_b)
    reduce(c0, rows_a)
    drain(rows_b, sem_b)

    @pl.when(c0 + 2 < NCH)
    def _():
      fire(c0 + 2, rows_a, sem_a)

    reduce(c0 + 1, rows_b)
    return 0

  lax.fori_loop(0, NCH // 2, pipe, 0)

  iota = lax.iota(jnp.int32, L)
  for g in range(EPW // L):
    bvec = g * L + iota
    pdt = [plsc.load_gather(acc, [bvec * D + d]) for d in range(D)]
    for o in range(O):
      lacc = bv[o, pl.ds(0, L)]
      for d in range(D):
        lacc = lacc + pdt[d] * wv[o, d, pl.ds(0, L)]
      sig = 1.0 / (1.0 + jnp.exp(-lacc))
      plsc.store_scatter(out_v, [bvec * O + o], sig)

  pltpu.sync_copy(out_v, out_hbm.at[pl.ds(base * O, EPW * O)])


def kernel(x, table, W, b):
  xf = x.reshape(-1).astype(jnp.int32)
  tT = table.T
  tail = table[NBF * 128:, :].reshape(-1)
  flat = _transpose_kernel(tT, tail)
  table_rm = flat.reshape(VOCAB, D)
  wrep = jnp.broadcast_to((W * (1.0 / H))[:, :, None], (O, D, L))
  brep = jnp.broadcast_to(b[:, None], (O, L))
  return _gather_kernel(xf, table_rm, wrep, brep).reshape(B, O)


# speedup vs baseline: 1.5248x; 1.5248x over previous
import functools

import jax
import jax.numpy as jnp
from jax import lax
from jax.experimental import pallas as pl
from jax.experimental.pallas import tpu as pltpu
from jax.experimental.pallas import tpu_sc as plsc

VOCAB = 1000000
D = 32
O = 6
B = 4096
H = 200

NC = 2
NS = 16
L = 16
NW = NC * NS
EPW = B // NW

NBF = VOCAB // 128
TAIL = VOCAB - NBF * 128
ROUNDS = (NBF + NW - 1) // NW

CHUNK = 4
NCH = EPW // CHUNK
GSUB = 200
NG = CHUNK * H // GSUB

_mesh = plsc.VectorSubcoreMesh(
    core_axis_name="c", subcore_axis_name="s", num_cores=NC, num_subcores=NS)


@functools.partial(
    pl.kernel,
    out_type=jax.ShapeDtypeStruct((VOCAB * D,), jnp.float32),
    mesh=_mesh,
    compiler_params=pltpu.CompilerParams(
        needs_layout_passes=False, use_tc_tiling_on_sc=True),
    scratch_types=dict(
        blk_a=pltpu.VMEM((D, 128), jnp.float32),
        blk_b=pltpu.VMEM((D, 128), jnp.float32),
        rows_a=pltpu.VMEM((128 * D,), jnp.float32),
        rows_b=pltpu.VMEM((128 * D,), jnp.float32),
        tail_v=pltpu.VMEM((TAIL * D,), jnp.float32),
        sem_a=pltpu.SemaphoreType.DMA,
        sem_b=pltpu.SemaphoreType.DMA,
        osem_a=pltpu.SemaphoreType.DMA,
        osem_b=pltpu.SemaphoreType.DMA,
    ),
)
def _transpose_kernel(tT_hbm, tail_hbm, out_hbm, blk_a, blk_b, rows_a,
                      rows_b, tail_v, sem_a, sem_b, osem_a, osem_b):
  wid = lax.axis_index("s") * NC + lax.axis_index("c")
  iota = lax.iota(jnp.int32, L)

  def fire(j, blk, sem):
    off = pl.multiple_of(j * 128, 128)
    pltpu.async_copy(tT_hbm.at[:, pl.ds(off, 128)], blk, sem)

  def drain_in(blk, sem):
    pltpu.make_async_copy(tT_hbm.at[:, pl.ds(0, 128)], blk, sem).wait()

  def shuffle(j, blk, rows, osem):
    @plsc.parallel_loop(0, 128, 1, unroll=8)
    def _(rl):
      rlv = jnp.full((L,), rl, jnp.int32)
      v0 = plsc.load_gather(blk, [iota, rlv])
      v1 = plsc.load_gather(blk, [iota + L, rlv])
      roff = pl.multiple_of(rl * D, 8)
      rows[pl.ds(roff, L)] = v0
      rows[pl.ds(roff + L, L)] = v1

    off = pl.multiple_of(j * (128 * D), 8)
    pltpu.async_copy(rows, out_hbm.at[pl.ds(off, 128 * D)], osem)

  def drain_out(rows, osem):
    pltpu.make_async_copy(rows, out_hbm.at[pl.ds(0, 128 * D)], osem).wait()

  fire(wid, blk_a, sem_a)

  def body(t, _):
    i0 = t * 2
    j0 = i0 * NW + wid
    j1 = j0 + NW
    j2 = j1 + NW
    drain_in(blk_a, sem_a)

    @pl.when(j1 < NBF)
    def _():
      fire(j1, blk_b, sem_b)

    @pl.when(t > 0)
    def _():
      drain_out(rows_a, osem_a)

    shuffle(j0, blk_a, rows_a, osem_a)

    @pl.when(j1 < NBF)
    def _():
      drain_in(blk_b, sem_b)

      @pl.when(j2 < NBF)
      def _():
        fire(j2, blk_a, sem_a)

      @pl.when(t > 0)
      def _():
        drain_out(rows_b, osem_b)

      shuffle(j1, blk_b, rows_b, osem_b)
    return 0

  nit = (ROUNDS + 1) // 2

  def guarded_body(t, c):
    @pl.when(t * 2 * NW + wid < NBF)
    def _():
      body(t, c)
    return 0

  lax.fori_loop(0, nit, guarded_body, 0)

  @pl.when(wid == 0)
  def _():
    pltpu.sync_copy(tail_hbm, tail_v)
    pltpu.sync_copy(tail_v, out_hbm.at[pl.ds(NBF * 128 * D, TAIL * D)])

  drain_out(rows_a, osem_a)
  drain_out(rows_b, osem_b)


@functools.partial(
    pl.kernel,
    out_type=jax.ShapeDtypeStruct((B * O,), jnp.float32),
    mesh=_mesh,
    compiler_params=pltpu.CompilerParams(
        needs_layout_passes=False, use_tc_tiling_on_sc=False),
    scratch_types=dict(
        idx_v=pltpu.VMEM((EPW * H,), jnp.int32),
        rows_a=pltpu.VMEM((CHUNK * H, D), jnp.float32),
        rows_b=pltpu.VMEM((CHUNK * H, D), jnp.float32),
        acc=pltpu.VMEM((EPW * D,), jnp.float32),
        wv=pltpu.VMEM((O, D, L), jnp.float32),
        bv=pltpu.VMEM((O, L), jnp.float32),
        out_v=pltpu.VMEM((EPW * O,), jnp.float32),
        sem_a=pltpu.SemaphoreType.DMA,
        sem_b=pltpu.SemaphoreType.DMA,
    ),
)
def _gather_kernel(x_hbm, table_hbm, w_hbm, b_hbm, out_hbm,
                   idx_v, rows_a, rows_b, acc, wv, bv, out_v, sem_a, sem_b):
  wid = lax.axis_index("s") * NC + lax.axis_index("c")
  base = wid * EPW

  pltpu.sync_copy(x_hbm.at[pl.ds(base * H, EPW * H)], idx_v)
  pltpu.sync_copy(w_hbm, wv)
  pltpu.sync_copy(b_hbm, bv)

  def fire(c, buf, sem):
    for j in range(NG):
      off = pl.multiple_of(c * (CHUNK * H) + j * GSUB, 8)
      pltpu.async_copy(
          table_hbm.at[idx_v.at[pl.ds(off, GSUB)]],
          buf.at[pl.ds(j * GSUB, GSUB)], sem)

  def drain(buf, sem):
    pltpu.make_async_copy(table_hbm.at[pl.ds(0, CHUNK * H)], buf, sem).wait()

  zeros = jnp.zeros((L,), jnp.float32)
  zeros8 = (zeros,) * 8

  def reduce(c, buf):
    for el in range(CHUNK):
      def body(i, carry, el=el):
        c0, c1, c2, c3, c4, c5, c6, c7 = carry
        r = el * H + i * 8
        for k in range(4):
          c0 = c0 + buf[r + 2 * k, pl.ds(0, L)]
          c1 = c1 + buf[r + 2 * k, pl.ds(L, L)]
          c2 = c2 + buf[r + 2 * k + 1, pl.ds(0, L)]
          c3 = c3 + buf[r + 2 * k + 1, pl.ds(L, L)]
        return (c2, c3, c4, c5, c6, c7, c0, c1)
      ch = lax.fori_loop(0, H // 8, body, zeros8)
      a0 = (ch[0] + ch[2]) + (ch[4] + ch[6])
      a1 = (ch[1] + ch[3]) + (ch[5] + ch[7])
      eoff = pl.multiple_of((c * CHUNK + el) * D, 8)
      acc[pl.ds(eoff, L)] = a0
      acc[pl.ds(eoff + L, L)] = a1

  fire(0, rows_a, sem_a)

  def pipe(ee, _):
    c0 = ee * 2
    drain(rows_a, sem_a)
    fire(c0 + 1, rows_b, sem_b)
    reduce(c0, rows_a)
    drain(rows_b, sem_b)

    @pl.when(c0 + 2 < NCH)
    def _():
      fire(c0 + 2, rows_a, sem_a)

    reduce(c0 + 1, rows_b)
    return 0

  lax.fori_loop(0, NCH // 2, pipe, 0)

  iota = lax.iota(jnp.int32, L)
  for g in range(EPW // L):
    bvec = g * L + iota
    pdt = [plsc.load_gather(acc, [bvec * D + d]) for d in range(D)]
    for o in range(O):
      lacc = bv[o, pl.ds(0, L)]
      for d in range(D):
        lacc = lacc + pdt[d] * wv[o, d, pl.ds(0, L)]
      sig = 1.0 / (1.0 + jnp.exp(-lacc))
      plsc.store_scatter(out_v, [bvec * O + o], sig)

  pltpu.sync_copy(out_v, out_hbm.at[pl.ds(base * O, EPW * O)])


def kernel(x, table, W, b):
  xf = x.reshape(-1).astype(jnp.int32)
  tT = table.T
  tail = table[NBF * 128:, :].reshape(-1)
  flat = _transpose_kernel(tT, tail)
  table_rm = flat.reshape(VOCAB, D)
  wrep = jnp.broadcast_to((W * (1.0 / H))[:, :, None], (O, D, L))
  brep = jnp.broadcast_to(b[:, None], (O, L))
  return _gather_kernel(xf, table_rm, wrep, brep).reshape(B, O)


# speedup vs baseline: 2.6105x; 1.7119x over previous
import functools

import jax
import jax.numpy as jnp
from jax import lax
from jax.experimental import pallas as pl
from jax.experimental.pallas import tpu as pltpu
from jax.experimental.pallas import tpu_sc as plsc

VOCAB = 1000000
D = 32
O = 6
B = 4096
H = 200

NC = 2
NS = 16
L = 16
NW = NC * NS
EPW = B // NW

NBF = VOCAB // 128
TAIL = VOCAB - NBF * 128
ROUNDS = (NBF + NW - 1) // NW

CHUNK = 4
NCH = EPW // CHUNK
GSUB = 200
NG = CHUNK * H // GSUB

_mesh = plsc.VectorSubcoreMesh(
    core_axis_name="c", subcore_axis_name="s", num_cores=NC, num_subcores=NS)


@functools.partial(
    pl.kernel,
    out_type=jax.ShapeDtypeStruct((VOCAB * D,), jnp.float32),
    mesh=_mesh,
    compiler_params=pltpu.CompilerParams(
        needs_layout_passes=False, use_tc_tiling_on_sc=True),
    scratch_types=dict(
        blk_a=pltpu.VMEM((D, 128), jnp.float32),
        blk_b=pltpu.VMEM((D, 128), jnp.float32),
        rows_a=pltpu.VMEM((128 * D,), jnp.float32),
        rows_b=pltpu.VMEM((128 * D,), jnp.float32),
        tail_v=pltpu.VMEM((TAIL * D,), jnp.float32),
        sem_a=pltpu.SemaphoreType.DMA,
        sem_b=pltpu.SemaphoreType.DMA,
        osem_a=pltpu.SemaphoreType.DMA,
        osem_b=pltpu.SemaphoreType.DMA,
    ),
)
def _transpose_kernel(tT_hbm, tail_hbm, out_hbm, blk_a, blk_b, rows_a,
                      rows_b, tail_v, sem_a, sem_b, osem_a, osem_b):
  wid = lax.axis_index("s") * NC + lax.axis_index("c")
  iota = lax.iota(jnp.int32, L)

  def fire(j, blk, sem):
    off = pl.multiple_of(j * 128, 128)
    pltpu.async_copy(tT_hbm.at[:, pl.ds(off, 128)], blk, sem)

  def drain_in(blk, sem):
    pltpu.make_async_copy(tT_hbm.at[:, pl.ds(0, 128)], blk, sem).wait()

  def shuffle(j, blk, rows, osem):
    @plsc.parallel_loop(0, 32 * L, 1, unroll=8)
    def _(t):
      s = t & (L - 1)
      g = t >> 4
      c0 = (g & 1) << 4
      rl0 = (g >> 1) << 4
      rot = (iota + s) & (L - 1)
      v = plsc.load_gather(blk, [c0 + iota, rl0 + rot])
      plsc.store_scatter(rows, [(rl0 * D + c0) + iota + rot * D], v)

    off = pl.multiple_of(j * (128 * D), 8)
    pltpu.async_copy(rows, out_hbm.at[pl.ds(off, 128 * D)], osem)

  def drain_out(rows, osem):
    pltpu.make_async_copy(rows, out_hbm.at[pl.ds(0, 128 * D)], osem).wait()

  fire(wid, blk_a, sem_a)

  def body(t, _):
    i0 = t * 2
    j0 = i0 * NW + wid
    j1 = j0 + NW
    j2 = j1 + NW
    drain_in(blk_a, sem_a)

    @pl.when(j1 < NBF)
    def _():
      fire(j1, blk_b, sem_b)

    @pl.when(t > 0)
    def _():
      drain_out(rows_a, osem_a)

    shuffle(j0, blk_a, rows_a, osem_a)

    @pl.when(j1 < NBF)
    def _():
      drain_in(blk_b, sem_b)

      @pl.when(j2 < NBF)
      def _():
        fire(j2, blk_a, sem_a)

      @pl.when(t > 0)
      def _():
        drain_out(rows_b, osem_b)

      shuffle(j1, blk_b, rows_b, osem_b)
    return 0

  nit = (ROUNDS + 1) // 2

  def guarded_body(t, c):
    @pl.when(t * 2 * NW + wid < NBF)
    def _():
      body(t, c)
    return 0

  lax.fori_loop(0, nit, guarded_body, 0)

  @pl.when(wid == 0)
  def _():
    pltpu.sync_copy(tail_hbm, tail_v)
    pltpu.sync_copy(tail_v, out_hbm.at[pl.ds(NBF * 128 * D, TAIL * D)])

  drain_out(rows_a, osem_a)
  drain_out(rows_b, osem_b)


@functools.partial(
    pl.kernel,
    out_type=jax.ShapeDtypeStruct((B * O,), jnp.float32),
    mesh=_mesh,
    compiler_params=pltpu.CompilerParams(
        needs_layout_passes=False, use_tc_tiling_on_sc=False),
    scratch_types=dict(
        idx_v=pltpu.VMEM((EPW * H,), jnp.int32),
        rows_a=pltpu.VMEM((CHUNK * H, D), jnp.float32),
        rows_b=pltpu.VMEM((CHUNK * H, D), jnp.float32),
        acc=pltpu.VMEM((EPW * D,), jnp.float32),
        wv=pltpu.VMEM((O, D, L), jnp.float32),
        bv=pltpu.VMEM((O, L), jnp.float32),
        out_v=pltpu.VMEM((EPW * O,), jnp.float32),
        sem_a=pltpu.SemaphoreType.DMA,
        sem_b=pltpu.SemaphoreType.DMA,
    ),
)
def _gather_kernel(x_hbm, table_hbm, w_hbm, b_hbm, out_hbm,
                   idx_v, rows_a, rows_b, acc, wv, bv, out_v, sem_a, sem_b):
  wid = lax.axis_index("s") * NC + lax.axis_index("c")
  base = wid * EPW

  pltpu.sync_copy(x_hbm.at[pl.ds(base * H, EPW * H)], idx_v)
  pltpu.sync_copy(w_hbm, wv)
  pltpu.sync_copy(b_hbm, bv)

  def fire(c, buf, sem):
    for j in range(NG):
      off = pl.multiple_of(c * (CHUNK * H) + j * GSUB, 8)
      pltpu.async_copy(
          table_hbm.at[idx_v.at[pl.ds(off, GSUB)]],
          buf.at[pl.ds(j * GSUB, GSUB)], sem)

  def drain(buf, sem):
    pltpu.make_async_copy(table_hbm.at[pl.ds(0, CHUNK * H)], buf, sem).wait()

  zeros = jnp.zeros((L,), jnp.float32)
  zeros8 = (zeros,) * 8

  def reduce(c, buf):
    for el in range(CHUNK):
      def body(i, carry, el=el):
        c0, c1, c2, c3, c4, c5, c6, c7 = carry
        r = el * H + i * 8
        for k in range(4):
          c0 = c0 + buf[r + 2 * k, pl.ds(0, L)]
          c1 = c1 + buf[r + 2 * k, pl.ds(L, L)]
          c2 = c2 + buf[r + 2 * k + 1, pl.ds(0, L)]
          c3 = c3 + buf[r + 2 * k + 1, pl.ds(L, L)]
        return (c2, c3, c4, c5, c6, c7, c0, c1)
      ch = lax.fori_loop(0, H // 8, body, zeros8)
      a0 = (ch[0] + ch[2]) + (ch[4] + ch[6])
      a1 = (ch[1] + ch[3]) + (ch[5] + ch[7])
      eoff = pl.multiple_of((c * CHUNK + el) * D, 8)
      acc[pl.ds(eoff, L)] = a0
      acc[pl.ds(eoff + L, L)] = a1

  fire(0, rows_a, sem_a)

  def pipe(ee, _):
    c0 = ee * 2
    drain(rows_a, sem_a)
    fire(c0 + 1, rows_b, sem_b)
    reduce(c0, rows_a)
    drain(rows_b, sem_b)

    @pl.when(c0 + 2 < NCH)
    def _():
      fire(c0 + 2, rows_a, sem_a)

    reduce(c0 + 1, rows_b)
    return 0

  lax.fori_loop(0, NCH // 2, pipe, 0)

  iota = lax.iota(jnp.int32, L)
  for g in range(EPW // L):
    bvec = g * L + iota
    pdt = [plsc.load_gather(acc, [bvec * D + d]) for d in range(D)]
    for o in range(O):
      lacc = bv[o, pl.ds(0, L)]
      for d in range(D):
        lacc = lacc + pdt[d] * wv[o, d, pl.ds(0, L)]
      sig = 1.0 / (1.0 + jnp.exp(-lacc))
      plsc.store_scatter(out_v, [bvec * O + o], sig)

  pltpu.sync_copy(out_v, out_hbm.at[pl.ds(base * O, EPW * O)])


def kernel(x, table, W, b):
  xf = x.reshape(-1).astype(jnp.int32)
  tT = table.T
  tail = table[NBF * 128:, :].reshape(-1)
  flat = _transpose_kernel(tT, tail)
  table_rm = flat.reshape(VOCAB, D)
  wrep = jnp.broadcast_to((W * (1.0 / H))[:, :, None], (O, D, L))
  brep = jnp.broadcast_to(b[:, None], (O, L))
  return _gather_kernel(xf, table_rm, wrep, brep).reshape(B, O)
